# baseline (speedup 1.0000x reference)
import functools

import jax
import jax.numpy as jnp
from jax import lax
from jax.experimental import pallas as pl
from jax.experimental.pallas import tpu as pltpu
from jax.experimental.pallas import tpu_sc as plsc

_LANES = 16
_CHUNK = 128
_ECH = 64
_NSC = 2
_NSUBC = 16


def _round_up(a, b):
    return (a + b - 1) // b * b


def _mm_body(x_ref, w_ref, o_ref):
    o_ref[...] = jnp.dot(x_ref[...], w_ref[...],
                         preferred_element_type=jnp.float32)


def _dis_body(deg_ref, o_ref):
    d = deg_ref[...]
    o_ref[...] = jnp.where(d > 0.0, lax.rsqrt(d), 0.0)


def kernel(x, edge_index, edge_weight, subG_node, W1, b1, gn_w, gn_b, gn_ms,
           W2, b2, Wm1, bm1, Wm2, bm2):
    N, C, D = x.shape
    E = edge_index.shape[1]
    NSUB, SUBSZ = subG_node.shape
    HID = W1.shape[1]
    OUT = W2.shape[1]
    f32 = jnp.float32
    FB = HID // _NSC
    NCB = FB // _LANES

    assert C == 2 and OUT == HID and HID % (2 * _LANES) == 0

    R = _round_up(_round_up(E, _ECH) // _ECH, 3 * _NSUBC)
    E_pad = R * _ECH
    RT = R // _NSUBC

    row_p = jnp.concatenate(
        [edge_index[0].astype(jnp.int32),
         jnp.zeros((E_pad - E,), jnp.int32)]).reshape(R, _ECH)
    col_p = jnp.concatenate(
        [edge_index[1].astype(jnp.int32),
         jnp.zeros((E_pad - E,), jnp.int32)]).reshape(R, _ECH)
    w_p = jnp.concatenate(
        [edge_weight.astype(f32),
         jnp.zeros((E_pad - E,), f32)]).reshape(R, _ECH)

    NP = _round_up(N, 2048)
    NPT = NP // _NSUBC
    NT = NP // _NSUBC
    NF = NT // _ECH

    dinit = jnp.concatenate(
        [jnp.ones((N,), f32), jnp.zeros((NP - N,), f32)])
    zeros_nf = jnp.zeros((NP, FB), f32)

    mesh1 = plsc.VectorSubcoreMesh(core_axis_name="c", subcore_axis_name="s",
                                   num_cores=1)
    mesh2 = plsc.VectorSubcoreMesh(core_axis_name="c", subcore_axis_name="s",
                                   num_cores=_NSC)
    _sc_params = pltpu.CompilerParams(needs_layout_passes=False,
                                      use_tc_tiling_on_sc=False)

    @functools.partial(
        pl.kernel,
        out_type=jax.ShapeDtypeStruct((NP,), f32),
        mesh=mesh1,
        compiler_params=_sc_params,
        scratch_types=[
            pltpu.VMEM((RT, _ECH), jnp.int32),
            pltpu.VMEM((RT, _ECH), f32),
            pltpu.VMEM((NPT,), f32),
            pltpu.VMEM_SHARED((NP,), f32),
        ],
    )
    def k_deg(col_hbm, w_hbm, dinit_hbm, deg_hbm, col_v, w_v, deg_v, acc_sh):
        s = lax.axis_index("s")
        pltpu.sync_copy(col_hbm.at[pl.ds(s * RT, RT)], col_v)
        pltpu.sync_copy(w_hbm.at[pl.ds(s * RT, RT)], w_v)
        pltpu.sync_copy(dinit_hbm.at[pl.ds(s * NPT, NPT)], deg_v)
        pltpu.sync_copy(deg_v, acc_sh.at[pl.ds(s * NPT, NPT)])
        plsc.subcore_barrier()

        def body(j, carry):
            pltpu.sync_copy(w_v.at[j], acc_sh.at[col_v.at[j]], add=True)
            return carry
        lax.fori_loop(0, RT, body, 0)
        plsc.subcore_barrier()
        pltpu.sync_copy(acc_sh.at[pl.ds(s * NPT, NPT)], deg_v)
        pltpu.sync_copy(deg_v, deg_hbm.at[pl.ds(s * NPT, NPT)])

    deg = k_deg(col_p, w_p, dinit)

    dis2d = pl.pallas_call(
        _dis_body,
        out_shape=jax.ShapeDtypeStruct((NP // 128, 128), f32),
    )(deg.reshape(NP // 128, 128))
    dis = dis2d.reshape(NP)

    xpad = jnp.concatenate(
        [x.transpose(1, 0, 2),
         jnp.zeros((C, NP - N, D), f32)], axis=1)
    xt_dup = jnp.concatenate(
        [xpad[0], xpad[0], xpad[1], xpad[1]]) if C == 2 else None
    w1s = jnp.stack([W1[:, b * FB:(b + 1) * FB] for b in range(_NSC)])

    def _mmb_body(x_ref, w_ref, o_ref):
        o_ref[...] = jnp.dot(x_ref[...], w_ref[0],
                             preferred_element_type=jnp.float32)

    h1 = pl.pallas_call(
        _mmb_body,
        grid=(C * _NSC,),
        in_specs=[
            pl.BlockSpec((NP, D), lambda g: (g, 0)),
            pl.BlockSpec((1, D, FB), lambda g: (g % _NSC, 0, 0)),
        ],
        out_specs=pl.BlockSpec((NP, FB), lambda g: (g, 0)),
        out_shape=jax.ShapeDtypeStruct((C * _NSC * NP, FB), f32),
    )(xt_dup, w1s)

    def make_conv(nepochs, nbufs):
        @functools.partial(
            pl.kernel,
            out_type=jax.ShapeDtypeStruct((nepochs * _NSC * NP, FB), f32),
            mesh=mesh2,
            compiler_params=_sc_params,
            scratch_types=[
                pltpu.VMEM((RT, _ECH), jnp.int32),
                pltpu.VMEM((RT, _ECH), jnp.int32),
                pltpu.VMEM((RT, _ECH), f32),
                pltpu.VMEM((_ECH, FB), f32),
                pltpu.VMEM((_ECH, FB), f32),
                pltpu.VMEM((_ECH, FB), f32),
                pltpu.VMEM((_ECH, FB), f32),
                pltpu.VMEM((_ECH, FB), f32),
                pltpu.VMEM((NP,), f32),
                pltpu.VMEM_SHARED((NP, FB), f32),
                pltpu.SemaphoreType.DMA,
                pltpu.SemaphoreType.DMA,
                pltpu.SemaphoreType.DMA,
                pltpu.SemaphoreType.DMA,
            ],
        )
        def k_conv(h_hbm, row_hbm, col_hbm, w_hbm, dis_hbm, z_hbm, o_hbm,
                   row_v, col_v, a_v, gbuf, gbuf1, gbuf2, gbuf3, hbuf, dis_v,
                   acc_sh, gsem0, gsem1, gsem2, gsem3):
            c = lax.axis_index("c")
            s = lax.axis_index("s")
            bufs = (gbuf, gbuf1, gbuf2, gbuf3)[:nbufs]
            sems = (gsem0, gsem1, gsem2, gsem3)[:nbufs]
            pltpu.sync_copy(row_hbm.at[pl.ds(s * RT, RT)], row_v)
            pltpu.sync_copy(col_hbm.at[pl.ds(s * RT, RT)], col_v)
            pltpu.sync_copy(w_hbm.at[pl.ds(s * RT, RT)], a_v)
            pltpu.sync_copy(dis_hbm, dis_v)

            def abody(jr, carry):
                for jc in range(_ECH // _LANES):
                    sl = pl.ds(jc * _LANES, _LANES)
                    r16 = row_v[jr, sl]
                    a_v[jr, sl] = a_v[jr, sl] * plsc.load_gather(dis_v, [r16])
                    row_v[jr, sl] = r16 + c * NP
                return carry
            lax.fori_loop(0, RT, abody, 0)

            def scale(j, buf):
                for g in range(_ECH // _LANES):
                    a16 = a_v[j, pl.ds(g * _LANES, _LANES)]
                    for l in range(_LANES):
                        av = jnp.broadcast_to(a16[l], (_LANES,))
                        e = g * _LANES + l
                        for cc in range(NCB):
                            sl = pl.ds(cc * _LANES, _LANES)
                            buf[e, sl] = buf[e, sl] * av

            def epoch_body(ep, carry0):
                @pl.when(ep > 0)
                def _():
                    def rbody(jr, carry):
                        for jc in range(_ECH // _LANES):
                            sl = pl.ds(jc * _LANES, _LANES)
                            row_v[jr, sl] = row_v[jr, sl] + _NSC * NP
                        return carry
                    lax.fori_loop(0, RT, rbody, 0)
                pltpu.sync_copy(z_hbm.at[pl.ds(s * NT, NT)],
                                acc_sh.at[pl.ds(s * NT, NT)])
                plsc.subcore_barrier()

                for b in range(nbufs):
                    pltpu.async_copy(h_hbm.at[row_v.at[b]], bufs[b], sems[b])

                def pbody(t, carry):
                    for b in range(nbufs):
                        j = nbufs * t + b
                        pltpu.make_async_copy(h_hbm.at[row_v.at[j]], bufs[b],
                                              sems[b]).wait()
                        scale(j, bufs[b])
                        pltpu.async_copy(bufs[b], acc_sh.at[col_v.at[j]],
                                         sems[b], add=True)
                        b2 = (b - 2) % nbufs
                        jm = j - 2

                        def recycle(b2=b2, jm=jm):
                            pltpu.make_async_copy(
                                bufs[b2], acc_sh.at[col_v.at[jm]],
                                sems[b2]).wait()
                            nx = jnp.minimum(jm + nbufs, RT - 1)
                            pltpu.async_copy(h_hbm.at[row_v.at[nx]],
                                             bufs[b2], sems[b2])
                        if b >= 2:
                            recycle()
                        else:
                            pl.when(t > 0)(recycle)
                    return carry
                lax.fori_loop(0, RT // nbufs, pbody, 0)
                for jc in (RT - 2, RT - 1):
                    pltpu.make_async_copy(bufs[jc % nbufs],
                                          acc_sh.at[col_v.at[jc]],
                                          sems[jc % nbufs]).wait()
                for jpos in range(RT - 2, RT):
                    if jpos - 2 + nbufs >= RT:
                        bd = (jpos - 2) % nbufs
                        pltpu.make_async_copy(h_hbm.at[row_v.at[RT - 1]],
                                              bufs[bd], sems[bd]).wait()
                plsc.subcore_barrier()

                hb = (ep * _NSC + c) * NP

                def kbody(k, carry):
                    base = s * NT + k * _ECH
                    pltpu.sync_copy(acc_sh.at[pl.ds(base, _ECH)], gbuf)
                    pltpu.sync_copy(h_hbm.at[pl.ds(hb + base, _ECH)], hbuf)

                    def gbody(g, c2):
                        d16 = dis_v[pl.ds(base + g * _LANES, _LANES)]
                        for l in range(_LANES):
                            d1 = jnp.broadcast_to(d16[l], (_LANES,))
                            d2 = d1 * d1
                            e = g * _LANES + l
                            for cc in range(NCB):
                                sl = pl.ds(cc * _LANES, _LANES)
                                gbuf[e, sl] = (d1 * gbuf[e, sl]
                                               + d2 * hbuf[e, sl])
                        return c2
                    lax.fori_loop(0, _ECH // _LANES, gbody, 0)
                    pltpu.sync_copy(gbuf, o_hbm.at[pl.ds(hb + base, _ECH)])
                    return carry
                lax.fori_loop(0, NF, kbody, 0)
                plsc.subcore_barrier()
                return carry0
            lax.fori_loop(0, nepochs, epoch_body, 0)
        return k_conv

    o1 = make_conv(C, 3)(h1, row_p, col_p, w_p, dis, zeros_nf)

    def _mid_body(o_ref, b1_ref, gnw_ref, gnb_ref, gnms_ref, w2_ref, out_ref):
        def gnorm(oc):
            oc = oc + b1_ref[...]
            m = jnp.mean(oc, axis=0, keepdims=True)
            ctr = oc - gnms_ref[...] * m
            v = jnp.mean(ctr * ctr, axis=0, keepdims=True)
            t = ctr * lax.rsqrt(v + 1e-5) * gnw_ref[...] + gnb_ref[...]
            return jnp.maximum(t, 0.0)

        def chan(ci):
            return jnp.concatenate(
                [o_ref[(2 * ci) * NP:(2 * ci) * NP + N],
                 o_ref[(2 * ci + 1) * NP:(2 * ci + 1) * NP + N]], axis=1)
        rbar = 0.5 * (gnorm(chan(0)) + gnorm(chan(1)))
        for b in range(_NSC):
            out_ref[b * NP:b * NP + N] = jnp.dot(
                rbar, w2_ref[:, b * FB:(b + 1) * FB],
                preferred_element_type=jnp.float32)
            out_ref[b * NP + N:(b + 1) * NP] = jnp.zeros((NP - N, FB),
                                                         jnp.float32)

    h2 = pl.pallas_call(
        _mid_body,
        out_shape=jax.ShapeDtypeStruct((_NSC * NP, FB), f32),
    )(o1, b1, gn_w, gn_b, gn_ms, W2)

    q = make_conv(1, 3)(h2, row_p, col_p, w_p, dis, zeros_nf)

    SGR = NSUB * SUBSZ // _ECH
    NW = _NSC * _NSUBC
    RPT = SGR // NW
    PSG = NSUB // NW
    SPR = _ECH // SUBSZ
    assert SGR % NW == 0 and NSUB % NW == 0

    sg_flat = subG_node.astype(jnp.int32).reshape(NW, RPT, _ECH)
    sgN_flat = sg_flat + NP

    @functools.partial(
        pl.kernel,
        out_type=jax.ShapeDtypeStruct((NSUB, OUT), f32),
        mesh=mesh2,
        compiler_params=_sc_params,
        scratch_types=[
            pltpu.VMEM((RPT, _ECH), jnp.int32),
            pltpu.VMEM((RPT, _ECH), jnp.int32),
            pltpu.VMEM((_ECH, FB), f32),
            pltpu.VMEM((_ECH, FB), f32),
            pltpu.VMEM((PSG, OUT), f32),
        ],
    )
    def k_pool(q_hbm, sg_hbm, sgN_hbm, out_hbm, sg_v, sgN_v, g0, g1, pbuf):
        c = lax.axis_index("c")
        s = lax.axis_index("s")
        t = c * _NSUBC + s
        pltpu.sync_copy(sg_hbm.at[t], sg_v)
        pltpu.sync_copy(sgN_hbm.at[t], sgN_v)

        def jbody(jj, carry0):
            pltpu.sync_copy(q_hbm.at[sg_v.at[jj]], g0)
            pltpu.sync_copy(q_hbm.at[sgN_v.at[jj]], g1)

            def pbody(sg, carry):
                for cc in range(NCB):
                    sl = pl.ds(cc * _LANES, _LANES)
                    slr = pl.ds(FB + cc * _LANES, _LANES)
                    acc0 = g0[sg * SUBSZ, sl]
                    acc1 = g1[sg * SUBSZ, sl]
                    for tt in range(1, SUBSZ):
                        acc0 = acc0 + g0[sg * SUBSZ + tt, sl]
                        acc1 = acc1 + g1[sg * SUBSZ + tt, sl]
                    pbuf[jj * SPR + sg, sl] = acc0
                    pbuf[jj * SPR + sg, slr] = acc1
                return carry
            lax.fori_loop(0, SPR, pbody, 0)
            return carry0
        lax.fori_loop(0, RPT, jbody, 0)
        pltpu.sync_copy(pbuf, out_hbm.at[pl.ds(t * PSG, PSG)])

    pooled = k_pool(q, sg_flat, sgN_flat)

    def _mlp_body(p_ref, b2_ref, wm1_ref, bm1_ref, wm2_ref, bm2_ref, out_ref):
        pre = p_ref[...] + float(SUBSZ) * b2_ref[...]
        h = jnp.dot(pre, wm1_ref[...], preferred_element_type=jnp.float32)
        h = jnp.maximum(h + bm1_ref[...], 0.0)
        out_ref[...] = jnp.dot(h, wm2_ref[...],
                               preferred_element_type=jnp.float32) + bm2_ref[...]

    out = pl.pallas_call(
        _mlp_body,
        out_shape=jax.ShapeDtypeStruct((NSUB, Wm2.shape[1]), f32),
    )(pooled, b2, Wm1, bm1, Wm2, bm2)
    return out


# speedup vs baseline: 1.4779x; 1.4779x over previous
import functools

import jax
import jax.numpy as jnp
from jax import lax
from jax.experimental import pallas as pl
from jax.experimental.pallas import tpu as pltpu
from jax.experimental.pallas import tpu_sc as plsc

_LANES = 16
_CHUNK = 128
_ECH = 64
_NSC = 2
_NSUBC = 16


def _round_up(a, b):
    return (a + b - 1) // b * b


def _mm_body(x_ref, w_ref, o_ref):
    o_ref[...] = jnp.dot(x_ref[...], w_ref[...],
                         preferred_element_type=jnp.float32)


def _dis_body(deg_ref, o_ref):
    d = deg_ref[...]
    o_ref[...] = jnp.where(d > 0.0, lax.rsqrt(d), 0.0)


def kernel(x, edge_index, edge_weight, subG_node, W1, b1, gn_w, gn_b, gn_ms,
           W2, b2, Wm1, bm1, Wm2, bm2):
    N, C, D = x.shape
    E = edge_index.shape[1]
    NSUB, SUBSZ = subG_node.shape
    HID = W1.shape[1]
    OUT = W2.shape[1]
    f32 = jnp.float32
    FB = HID // _NSC
    NCB = FB // _LANES

    assert C == 2 and OUT == HID and HID % (2 * _LANES) == 0

    R = _round_up(_round_up(E, _ECH) // _ECH, 3 * _NSUBC)
    E_pad = R * _ECH
    RT = R // _NSUBC

    row_p = jnp.concatenate(
        [edge_index[0].astype(jnp.int32),
         jnp.zeros((E_pad - E,), jnp.int32)]).reshape(R, _ECH)
    col_p = jnp.concatenate(
        [edge_index[1].astype(jnp.int32),
         jnp.zeros((E_pad - E,), jnp.int32)]).reshape(R, _ECH)
    w_p = jnp.concatenate(
        [edge_weight.astype(f32),
         jnp.zeros((E_pad - E,), f32)]).reshape(R, _ECH)

    NP = _round_up(N, 2048)
    NPT = NP // _NSUBC
    NT = NP // _NSUBC
    NF = NT // _ECH

    dinit = jnp.concatenate(
        [jnp.ones((N,), f32), jnp.zeros((NP - N,), f32)])
    zeros_nf = jnp.zeros((NP, FB), f32)

    mesh1 = plsc.VectorSubcoreMesh(core_axis_name="c", subcore_axis_name="s",
                                   num_cores=1)
    mesh2 = plsc.VectorSubcoreMesh(core_axis_name="c", subcore_axis_name="s",
                                   num_cores=_NSC)
    _sc_params = pltpu.CompilerParams(needs_layout_passes=False,
                                      use_tc_tiling_on_sc=False)

    @functools.partial(
        pl.kernel,
        out_type=jax.ShapeDtypeStruct((NP,), f32),
        mesh=mesh1,
        compiler_params=_sc_params,
        scratch_types=[
            pltpu.VMEM((RT, _ECH), jnp.int32),
            pltpu.VMEM((RT, _ECH), f32),
            pltpu.VMEM((NPT,), f32),
            pltpu.VMEM_SHARED((NP,), f32),
        ],
    )
    def k_deg(col_hbm, w_hbm, dinit_hbm, deg_hbm, col_v, w_v, deg_v, acc_sh):
        s = lax.axis_index("s")
        pltpu.sync_copy(col_hbm.at[pl.ds(s * RT, RT)], col_v)
        pltpu.sync_copy(w_hbm.at[pl.ds(s * RT, RT)], w_v)
        pltpu.sync_copy(dinit_hbm.at[pl.ds(s * NPT, NPT)], deg_v)
        pltpu.sync_copy(deg_v, acc_sh.at[pl.ds(s * NPT, NPT)])
        plsc.subcore_barrier()

        def body(j, carry):
            pltpu.sync_copy(w_v.at[j], acc_sh.at[col_v.at[j]], add=True)
            return carry
        lax.fori_loop(0, RT, body, 0)
        plsc.subcore_barrier()
        pltpu.sync_copy(acc_sh.at[pl.ds(s * NPT, NPT)], deg_v)
        pltpu.sync_copy(deg_v, deg_hbm.at[pl.ds(s * NPT, NPT)])

    deg = k_deg(col_p, w_p, dinit)

    dis2d = pl.pallas_call(
        _dis_body,
        out_shape=jax.ShapeDtypeStruct((NP // 128, 128), f32),
    )(deg.reshape(NP // 128, 128))
    dis = dis2d.reshape(NP)

    xpad = jnp.concatenate(
        [x.transpose(1, 0, 2),
         jnp.zeros((C, NP - N, D), f32)], axis=1)
    xt_dup = jnp.concatenate(
        [xpad[0], xpad[0], xpad[1], xpad[1]]) if C == 2 else None
    w1s = jnp.stack([W1[:, b * FB:(b + 1) * FB] for b in range(_NSC)])

    def _mmb_body(x_ref, w_ref, o_ref):
        o_ref[...] = jnp.dot(x_ref[...], w_ref[0],
                             preferred_element_type=jnp.float32)

    h1 = pl.pallas_call(
        _mmb_body,
        grid=(C * _NSC,),
        in_specs=[
            pl.BlockSpec((NP, D), lambda g: (g, 0)),
            pl.BlockSpec((1, D, FB), lambda g: (g % _NSC, 0, 0)),
        ],
        out_specs=pl.BlockSpec((NP, FB), lambda g: (g, 0)),
        out_shape=jax.ShapeDtypeStruct((C * _NSC * NP, FB), f32),
    )(xt_dup, w1s)

    def make_conv(nepochs, nbufs):
        @functools.partial(
            pl.kernel,
            out_type=jax.ShapeDtypeStruct((nepochs * _NSC * NP, FB), f32),
            mesh=mesh2,
            compiler_params=_sc_params,
            scratch_types=[
                pltpu.VMEM((RT, _ECH), jnp.int32),
                pltpu.VMEM((RT, _ECH), jnp.int32),
                pltpu.VMEM((RT, _ECH), f32),
                pltpu.VMEM((_ECH, FB), f32),
                pltpu.VMEM((_ECH, FB), f32),
                pltpu.VMEM((_ECH, FB), f32),
                pltpu.VMEM((_ECH, FB), f32),
                pltpu.VMEM((_ECH, FB), f32),
                pltpu.VMEM((NP,), f32),
                pltpu.VMEM_SHARED((NP, FB), f32),
                pltpu.SemaphoreType.DMA,
                pltpu.SemaphoreType.DMA,
                pltpu.SemaphoreType.DMA,
                pltpu.SemaphoreType.DMA,
            ],
        )
        def k_conv(h_hbm, row_hbm, col_hbm, w_hbm, dis_hbm, z_hbm, o_hbm,
                   row_v, col_v, a_v, gbuf, gbuf1, gbuf2, gbuf3, hbuf, dis_v,
                   acc_sh, gsem0, gsem1, gsem2, gsem3):
            c = lax.axis_index("c")
            s = lax.axis_index("s")
            bufs = (gbuf, gbuf1, gbuf2, gbuf3)[:nbufs]
            sems = (gsem0, gsem1, gsem2, gsem3)[:nbufs]
            pltpu.sync_copy(row_hbm.at[pl.ds(s * RT, RT)], row_v)
            pltpu.sync_copy(col_hbm.at[pl.ds(s * RT, RT)], col_v)
            pltpu.sync_copy(w_hbm.at[pl.ds(s * RT, RT)], a_v)
            pltpu.sync_copy(dis_hbm, dis_v)

            def abody(jr, carry):
                for jc in range(_ECH // _LANES):
                    sl = pl.ds(jc * _LANES, _LANES)
                    r16 = row_v[jr, sl]
                    a_v[jr, sl] = a_v[jr, sl] * plsc.load_gather(dis_v, [r16])
                    row_v[jr, sl] = r16 + c * NP
                return carry
            lax.fori_loop(0, RT, abody, 0)

            def scale(j, buf):
                for g in range(_ECH // _LANES):
                    a16 = a_v[j, pl.ds(g * _LANES, _LANES)]
                    for l in range(_LANES):
                        av = jnp.broadcast_to(a16[l], (_LANES,))
                        e = g * _LANES + l
                        for cc in range(NCB):
                            sl = pl.ds(cc * _LANES, _LANES)
                            buf[e, sl] = buf[e, sl] * av

            def epoch_body(ep, carry0):
                @pl.when(ep > 0)
                def _():
                    def rbody(jr, carry):
                        for jc in range(_ECH // _LANES):
                            sl = pl.ds(jc * _LANES, _LANES)
                            row_v[jr, sl] = row_v[jr, sl] + _NSC * NP
                        return carry
                    lax.fori_loop(0, RT, rbody, 0)
                pltpu.sync_copy(z_hbm.at[pl.ds(s * NT, NT)],
                                acc_sh.at[pl.ds(s * NT, NT)])
                plsc.subcore_barrier()

                for b in range(nbufs):
                    pltpu.async_copy(h_hbm.at[row_v.at[b]], bufs[b], sems[b])

                def pbody(t, carry):
                    for b in range(nbufs):
                        j = nbufs * t + b
                        pltpu.make_async_copy(h_hbm.at[row_v.at[j]], bufs[b],
                                              sems[b]).wait()
                        scale(j, bufs[b])
                        pltpu.async_copy(bufs[b], acc_sh.at[col_v.at[j]],
                                         sems[b], add=True)
                        lag = nbufs - 2
                        b2 = (b - lag) % nbufs
                        jm = j - lag

                        def recycle(b2=b2, jm=jm):
                            pltpu.make_async_copy(
                                bufs[b2], acc_sh.at[col_v.at[jm]],
                                sems[b2]).wait()
                            nx = jnp.minimum(jm + nbufs, RT - 1)
                            pltpu.async_copy(h_hbm.at[row_v.at[nx]],
                                             bufs[b2], sems[b2])
                        if b >= lag:
                            recycle()
                        else:
                            pl.when(t > 0)(recycle)
                    return carry
                lax.fori_loop(0, RT // nbufs, pbody, 0)
                for jc in range(RT - (nbufs - 2), RT):
                    pltpu.make_async_copy(bufs[jc % nbufs],
                                          acc_sh.at[col_v.at[jc]],
                                          sems[jc % nbufs]).wait()
                for jpos in range(RT - 2, RT):
                    bd = (jpos - (nbufs - 2)) % nbufs
                    pltpu.make_async_copy(h_hbm.at[row_v.at[RT - 1]],
                                          bufs[bd], sems[bd]).wait()
                plsc.subcore_barrier()

                hb = (ep * _NSC + c) * NP

                def kbody(k, carry):
                    base = s * NT + k * _ECH
                    pltpu.sync_copy(acc_sh.at[pl.ds(base, _ECH)], gbuf)
                    pltpu.sync_copy(h_hbm.at[pl.ds(hb + base, _ECH)], hbuf)

                    def gbody(g, c2):
                        d16 = dis_v[pl.ds(base + g * _LANES, _LANES)]
                        for l in range(_LANES):
                            d1 = jnp.broadcast_to(d16[l], (_LANES,))
                            d2 = d1 * d1
                            e = g * _LANES + l
                            for cc in range(NCB):
                                sl = pl.ds(cc * _LANES, _LANES)
                                gbuf[e, sl] = (d1 * gbuf[e, sl]
                                               + d2 * hbuf[e, sl])
                        return c2
                    lax.fori_loop(0, _ECH // _LANES, gbody, 0)
                    pltpu.sync_copy(gbuf, o_hbm.at[pl.ds(hb + base, _ECH)])
                    return carry
                lax.fori_loop(0, NF, kbody, 0)
                plsc.subcore_barrier()
                return carry0
            lax.fori_loop(0, nepochs, epoch_body, 0)
        return k_conv

    o1 = make_conv(C, 3)(h1, row_p, col_p, w_p, dis, zeros_nf)

    def _mid_body(o_ref, b1_ref, gnw_ref, gnb_ref, gnms_ref, w2_ref, out_ref):
        def gnorm(oc):
            oc = oc + b1_ref[...]
            m = jnp.mean(oc, axis=0, keepdims=True)
            ctr = oc - gnms_ref[...] * m
            v = jnp.mean(ctr * ctr, axis=0, keepdims=True)
            t = ctr * lax.rsqrt(v + 1e-5) * gnw_ref[...] + gnb_ref[...]
            return jnp.maximum(t, 0.0)

        def chan(ci):
            return jnp.concatenate(
                [o_ref[(2 * ci) * NP:(2 * ci) * NP + N],
                 o_ref[(2 * ci + 1) * NP:(2 * ci + 1) * NP + N]], axis=1)
        rbar = 0.5 * (gnorm(chan(0)) + gnorm(chan(1)))
        for b in range(_NSC):
            out_ref[b * NP:b * NP + N] = jnp.dot(
                rbar, w2_ref[:, b * FB:(b + 1) * FB],
                preferred_element_type=jnp.float32)
            out_ref[b * NP + N:(b + 1) * NP] = jnp.zeros((NP - N, FB),
                                                         jnp.float32)

    h2 = pl.pallas_call(
        _mid_body,
        out_shape=jax.ShapeDtypeStruct((_NSC * NP, FB), f32),
    )(o1, b1, gn_w, gn_b, gn_ms, W2)

    q = make_conv(1, 3)(h2, row_p, col_p, w_p, dis, zeros_nf)

    SGR = NSUB * SUBSZ // _ECH
    NW = _NSC * _NSUBC
    RPT = SGR // NW
    PSG = NSUB // NW
    SPR = _ECH // SUBSZ
    assert SGR % NW == 0 and NSUB % NW == 0

    sg_flat = subG_node.astype(jnp.int32).reshape(NW, RPT, _ECH)
    sgN_flat = sg_flat + NP

    @functools.partial(
        pl.kernel,
        out_type=jax.ShapeDtypeStruct((NSUB, OUT), f32),
        mesh=mesh2,
        compiler_params=_sc_params,
        scratch_types=[
            pltpu.VMEM((RPT, _ECH), jnp.int32),
            pltpu.VMEM((RPT, _ECH), jnp.int32),
            pltpu.VMEM((_ECH, FB), f32),
            pltpu.VMEM((_ECH, FB), f32),
            pltpu.VMEM((PSG, OUT), f32),
        ],
    )
    def k_pool(q_hbm, sg_hbm, sgN_hbm, out_hbm, sg_v, sgN_v, g0, g1, pbuf):
        c = lax.axis_index("c")
        s = lax.axis_index("s")
        t = c * _NSUBC + s
        pltpu.sync_copy(sg_hbm.at[t], sg_v)
        pltpu.sync_copy(sgN_hbm.at[t], sgN_v)

        def jbody(jj, carry0):
            pltpu.sync_copy(q_hbm.at[sg_v.at[jj]], g0)
            pltpu.sync_copy(q_hbm.at[sgN_v.at[jj]], g1)

            def pbody(sg, carry):
                for cc in range(NCB):
                    sl = pl.ds(cc * _LANES, _LANES)
                    slr = pl.ds(FB + cc * _LANES, _LANES)
                    acc0 = g0[sg * SUBSZ, sl]
                    acc1 = g1[sg * SUBSZ, sl]
                    for tt in range(1, SUBSZ):
                        acc0 = acc0 + g0[sg * SUBSZ + tt, sl]
                        acc1 = acc1 + g1[sg * SUBSZ + tt, sl]
                    pbuf[jj * SPR + sg, sl] = acc0
                    pbuf[jj * SPR + sg, slr] = acc1
                return carry
            lax.fori_loop(0, SPR, pbody, 0)
            return carry0
        lax.fori_loop(0, RPT, jbody, 0)
        pltpu.sync_copy(pbuf, out_hbm.at[pl.ds(t * PSG, PSG)])

    pooled = k_pool(q, sg_flat, sgN_flat)

    def _mlp_body(p_ref, b2_ref, wm1_ref, bm1_ref, wm2_ref, bm2_ref, out_ref):
        pre = p_ref[...] + float(SUBSZ) * b2_ref[...]
        h = jnp.dot(pre, wm1_ref[...], preferred_element_type=jnp.float32)
        h = jnp.maximum(h + bm1_ref[...], 0.0)
        out_ref[...] = jnp.dot(h, wm2_ref[...],
                               preferred_element_type=jnp.float32) + bm2_ref[...]

    out = pl.pallas_call(
        _mlp_body,
        out_shape=jax.ShapeDtypeStruct((NSUB, Wm2.shape[1]), f32),
    )(pooled, b2, Wm1, bm1, Wm2, bm2)
    return out
